# baseline (speedup 1.0000x reference)
import functools

import jax
import jax.numpy as jnp
from jax import lax
from jax.experimental import pallas as pl
from jax.experimental.pallas import tpu as pltpu
from jax.experimental.pallas import tpu_sc as plsc

MAX_LEN = 8192
D_MODEL = 1024

_info = plsc.get_sparse_core_info()
NC, NS = _info.num_cores, _info.num_subcores
NW = NC * NS

B_TOTAL = 4 * 8192
B_PER_W = B_TOTAL // NW
CHUNK = 32
NCHUNK = B_PER_W // CHUNK
NBUF = 3


@functools.partial(
    pl.kernel,
    mesh=plsc.VectorSubcoreMesh(core_axis_name="c", subcore_axis_name="s"),
    out_type=jax.ShapeDtypeStruct((B_TOTAL, D_MODEL), jnp.float32),
    scratch_types=[
        pltpu.VMEM((NCHUNK, CHUNK), jnp.int32),
        pltpu.VMEM((NBUF, CHUNK, D_MODEL), jnp.float32),
        pltpu.SemaphoreType.DMA,
        pltpu.SemaphoreType.DMA,
    ],
)
def _emb_lookup(idx_hbm, table_hbm, out_hbm, idx_v, buf_v, gsem, ssem):
    wid = lax.axis_index("s") * NC + lax.axis_index("c")
    base = wid * B_PER_W
    pltpu.sync_copy(idx_hbm.at[wid], idx_v)

    def gather_start(j):
        pltpu.async_copy(table_hbm.at[idx_v.at[j]], buf_v.at[j % NBUF], gsem)

    def gather_wait():
        pltpu.make_async_copy(
            table_hbm.at[pl.ds(0, CHUNK)], buf_v.at[0], gsem
        ).wait()

    def scatter_start(j):
        pltpu.async_copy(
            buf_v.at[j % NBUF], out_hbm.at[pl.ds(base + j * CHUNK, CHUNK)], ssem
        )

    def scatter_wait():
        pltpu.make_async_copy(
            buf_v.at[0], out_hbm.at[pl.ds(base, CHUNK)], ssem
        ).wait()

    for b in range(NBUF - 1):
        gather_start(b)

    gather_wait()
    scatter_start(0)
    gather_start(NBUF - 1)

    def steady(j, carry):
        gather_wait()
        scatter_start(j)
        scatter_wait()
        gather_start(j + NBUF - 1)
        return carry

    lax.fori_loop(1, NCHUNK - NBUF + 1, steady, 0)

    for j in range(NCHUNK - NBUF + 1, NCHUNK):
        gather_wait()
        scatter_start(j)
    for _ in range(NBUF):
        scatter_wait()


def kernel(positions, pe_weight):
    idx = positions.reshape(NW, NCHUNK, CHUNK).astype(jnp.int32)
    out = _emb_lookup(idx, pe_weight)
    return out.reshape(positions.shape + (D_MODEL,))
